# initial kernel scaffold (unmeasured)
import jax
import jax.numpy as jnp
from jax import lax
from jax.experimental import pallas as pl
from jax.experimental.pallas import tpu as pltpu

N_DEV = 32
LOG2 = 5
B, SQ, D = 2, 128, 512
HQ, DH = 8, 64
SKV = 128
ROWS = B * SQ

_ROFFS = [0, 128, 192, 224, 240]


def kernel(x, Wq, Wo, K_ext, V_ext):
    def body(x_ref, wq_ref, wo_ref, k_hbm, v_hbm, out_ref,
             acc_ref, recv_ref, kv_ref, vv_ref, attn_ref,
             send_sems, recv_sems, copy_sems):
        me = lax.axis_index("i")

        kcopy = pltpu.make_async_copy(
            k_hbm.at[:, :, pl.ds(me * HQ, HQ), :], kv_ref, copy_sems.at[0])
        vcopy = pltpu.make_async_copy(
            v_hbm.at[:, :, pl.ds(me * HQ, HQ), :], vv_ref, copy_sems.at[1])
        kcopy.start()
        vcopy.start()

        barrier = pltpu.get_barrier_semaphore()
        for t in range(LOG2):
            pl.semaphore_signal(
                barrier, inc=1,
                device_id=(me ^ (1 << t),),
                device_id_type=pl.DeviceIdType.MESH)
        pl.semaphore_wait(barrier, LOG2)

        xv = x_ref[...].reshape(ROWS, D).astype(jnp.bfloat16)
        q = jnp.dot(xv, wq_ref[...].astype(jnp.bfloat16),
                    preferred_element_type=jnp.float32)

        kcopy.wait()
        vcopy.wait()

        for b in range(B):
            for h in range(HQ):
                qbh = q[b*SQ:(b+1)*SQ, h*DH:(h+1)*DH].astype(jnp.bfloat16)
                kbh = kv_ref[b, :, h, :].astype(jnp.bfloat16)
                vbh = vv_ref[b, :, h, :].astype(jnp.bfloat16)
                s = lax.dot_general(
                    qbh, kbh, (((1,), (1,)), ((), ())),
                    preferred_element_type=jnp.float32) * 0.125
                m = jnp.max(s, axis=1, keepdims=True)
                p = jnp.exp(s - m)
                l = jnp.sum(p, axis=1, keepdims=True)
                o = lax.dot_general(
                    p.astype(jnp.bfloat16), vbh, (((1,), (0,)), ((), ())),
                    preferred_element_type=jnp.float32)
                attn_ref[b*SQ:(b+1)*SQ, h*DH:(h+1)*DH] = (
                    (o / l).astype(jnp.bfloat16))

        acc_ref[...] = jnp.dot(
            attn_ref[...], wo_ref[...].astype(jnp.bfloat16),
            preferred_element_type=jnp.float32)

        base = me * 0
        for s in range(LOG2):
            half = (ROWS >> s) // 2
            dist = 1 << (LOG2 - 1 - s)
            partner = me ^ dist
            bit = (me >> (LOG2 - 1 - s)) & 1
            keep = base + bit * half
            send = base + (1 - bit) * half
            rdma = pltpu.make_async_remote_copy(
                src_ref=acc_ref.at[pl.ds(send, half)],
                dst_ref=recv_ref.at[pl.ds(_ROFFS[s], half)],
                send_sem=send_sems.at[s],
                recv_sem=recv_sems.at[s],
                device_id=(partner,),
                device_id_type=pl.DeviceIdType.MESH)
            rdma.start()
            rdma.wait()
            acc_ref[pl.ds(keep, half), :] = (
                acc_ref[pl.ds(keep, half), :]
                + recv_ref[pl.ds(_ROFFS[s], half), :])
            base = keep

        rb = me * HQ
        for t in range(LOG2):
            sz = HQ << t
            rdma = pltpu.make_async_remote_copy(
                src_ref=acc_ref.at[pl.ds(rb, sz)],
                dst_ref=acc_ref.at[pl.ds(rb, sz)],
                send_sem=send_sems.at[LOG2 + t],
                recv_sem=recv_sems.at[LOG2 + t],
                device_id=(me ^ (1 << t),),
                device_id_type=pl.DeviceIdType.MESH)
            rdma.start()
            rdma.wait()
            rb = rb & ~(HQ << t)

        out_ref[...] = acc_ref[...].reshape(B, SQ, D)

    return pl.pallas_call(
        body,
        out_shape=jax.ShapeDtypeStruct((B, SQ, D), jnp.float32),
        in_specs=[
            pl.BlockSpec(memory_space=pltpu.VMEM),
            pl.BlockSpec(memory_space=pltpu.VMEM),
            pl.BlockSpec(memory_space=pltpu.VMEM),
            pl.BlockSpec(memory_space=pltpu.ANY),
            pl.BlockSpec(memory_space=pltpu.ANY),
        ],
        out_specs=pl.BlockSpec(memory_space=pltpu.VMEM),
        scratch_shapes=[
            pltpu.VMEM((ROWS, D), jnp.float32),
            pltpu.VMEM((ROWS, D), jnp.float32),
            pltpu.VMEM((B, SKV, HQ, DH), jnp.float32),
            pltpu.VMEM((B, SKV, HQ, DH), jnp.float32),
            pltpu.VMEM((ROWS, D), jnp.bfloat16),
            pltpu.SemaphoreType.DMA((2 * LOG2,)),
            pltpu.SemaphoreType.DMA((2 * LOG2,)),
            pltpu.SemaphoreType.DMA((2,)),
        ],
        compiler_params=pltpu.CompilerParams(collective_id=0),
    )(x, Wq, Wo, K_ext, V_ext)


# baseline (device time: 89175 ns/iter reference)
import jax
import jax.numpy as jnp
from jax import lax
from jax.experimental import pallas as pl
from jax.experimental.pallas import tpu as pltpu

N_DEV = 32
LOG2 = 5
B, SQ, D = 2, 128, 512
HQ, DH = 8, 64
SKV = 128
ROWS = B * SQ

_ROFFS = [0, 128, 192, 224, 240]


def kernel(x, Wq, Wo, K_ext, V_ext):
    def body(x_ref, wq_ref, wo_ref, k_hbm, v_hbm, out_ref,
             acc_ref, recv_ref, kv_ref, vv_ref, attn_ref,
             send_sems, recv_sems, copy_sems):
        me = lax.axis_index("i")

        kcopy = pltpu.make_async_copy(
            k_hbm.at[:, :, pl.ds(me * HQ, HQ), :], kv_ref, copy_sems.at[0])
        vcopy = pltpu.make_async_copy(
            v_hbm.at[:, :, pl.ds(me * HQ, HQ), :], vv_ref, copy_sems.at[1])
        kcopy.start()
        vcopy.start()

        barrier = pltpu.get_barrier_semaphore()
        for t in range(LOG2):
            pl.semaphore_signal(
                barrier, inc=1,
                device_id=(me ^ (1 << t),),
                device_id_type=pl.DeviceIdType.MESH)
        pl.semaphore_wait(barrier, LOG2)

        xv = x_ref[...].reshape(ROWS, D).astype(jnp.bfloat16)
        q = jnp.dot(xv, wq_ref[...].astype(jnp.bfloat16),
                    preferred_element_type=jnp.float32)

        kcopy.wait()
        vcopy.wait()

        for b in range(B):
            for h in range(HQ):
                qbh = q[b*SQ:(b+1)*SQ, h*DH:(h+1)*DH].astype(jnp.bfloat16)
                kbh = kv_ref[b, :, h, :].astype(jnp.bfloat16)
                vbh = vv_ref[b, :, h, :].astype(jnp.bfloat16)
                s = lax.dot_general(
                    qbh, kbh, (((1,), (1,)), ((), ())),
                    preferred_element_type=jnp.float32) * 0.125
                m = jnp.max(s, axis=1, keepdims=True)
                p = jnp.exp(s - m)
                l = jnp.sum(p, axis=1, keepdims=True)
                o = lax.dot_general(
                    p.astype(jnp.bfloat16), vbh, (((1,), (0,)), ((), ())),
                    preferred_element_type=jnp.float32)
                attn_ref[b*SQ:(b+1)*SQ, h*DH:(h+1)*DH] = (
                    (o / l).astype(jnp.bfloat16))

        acc_ref[...] = jnp.dot(
            attn_ref[...], wo_ref[...].astype(jnp.bfloat16),
            preferred_element_type=jnp.float32)

        base = me * 0
        for s in range(LOG2):
            half = (ROWS >> s) // 2
            dist = 1 << (LOG2 - 1 - s)
            partner = me ^ dist
            bit = (me >> (LOG2 - 1 - s)) & 1
            keep = pl.multiple_of(base + bit * half, 8)
            send = pl.multiple_of(base + (1 - bit) * half, 8)
            rdma = pltpu.make_async_remote_copy(
                src_ref=acc_ref.at[pl.ds(send, half)],
                dst_ref=recv_ref.at[pl.ds(_ROFFS[s], half)],
                send_sem=send_sems.at[s],
                recv_sem=recv_sems.at[s],
                device_id=(partner,),
                device_id_type=pl.DeviceIdType.MESH)
            rdma.start()
            rdma.wait()
            acc_ref[pl.ds(keep, half), :] = (
                acc_ref[pl.ds(keep, half), :]
                + recv_ref[pl.ds(_ROFFS[s], half), :])
            base = keep

        rb = me * HQ
        for t in range(LOG2):
            sz = HQ << t
            rbm = pl.multiple_of(rb, 8)
            rdma = pltpu.make_async_remote_copy(
                src_ref=acc_ref.at[pl.ds(rbm, sz)],
                dst_ref=acc_ref.at[pl.ds(rbm, sz)],
                send_sem=send_sems.at[LOG2 + t],
                recv_sem=recv_sems.at[LOG2 + t],
                device_id=(me ^ (1 << t),),
                device_id_type=pl.DeviceIdType.MESH)
            rdma.start()
            rdma.wait()
            rb = rb & ~(HQ << t)

        out_ref[...] = acc_ref[...].reshape(B, SQ, D)

    return pl.pallas_call(
        body,
        out_shape=jax.ShapeDtypeStruct((B, SQ, D), jnp.float32),
        in_specs=[
            pl.BlockSpec(memory_space=pltpu.VMEM),
            pl.BlockSpec(memory_space=pltpu.VMEM),
            pl.BlockSpec(memory_space=pltpu.VMEM),
            pl.BlockSpec(memory_space=pl.ANY),
            pl.BlockSpec(memory_space=pl.ANY),
        ],
        out_specs=pl.BlockSpec(memory_space=pltpu.VMEM),
        scratch_shapes=[
            pltpu.VMEM((ROWS, D), jnp.float32),
            pltpu.VMEM((ROWS, D), jnp.float32),
            pltpu.VMEM((B, SKV, HQ, DH), jnp.float32),
            pltpu.VMEM((B, SKV, HQ, DH), jnp.float32),
            pltpu.VMEM((ROWS, D), jnp.bfloat16),
            pltpu.SemaphoreType.DMA((2 * LOG2,)),
            pltpu.SemaphoreType.DMA((2 * LOG2,)),
            pltpu.SemaphoreType.DMA((2,)),
        ],
        compiler_params=pltpu.CompilerParams(collective_id=0),
    )(x, Wq, Wo, K_ext, V_ext)


# device time: 58946 ns/iter; 1.5128x vs baseline; 1.5128x over previous
import jax
import jax.numpy as jnp
from jax import lax
from jax.experimental import pallas as pl
from jax.experimental.pallas import tpu as pltpu

N_DEV = 32
B, SQ, D = 2, 128, 512
HQ, DH = 8, 64
SKV = 128
ROWS = B * SQ
SEG = ROWS // N_DEV


def kernel(x, Wq, Wo, K_ext, V_ext):
    def body(x_ref, wq_ref, wo_ref, k_hbm, v_hbm, out_ref,
             part_ref, rs_ref, stage_ref, kv_ref, vv_ref, attn_ref,
             s1_send, s1_recv, s2_send, s2_recv, copy_sems):
        me = lax.axis_index("i")

        kcopy = pltpu.make_async_copy(
            k_hbm.at[:, :, pl.ds(me * HQ, HQ), :], kv_ref, copy_sems.at[0])
        vcopy = pltpu.make_async_copy(
            v_hbm.at[:, :, pl.ds(me * HQ, HQ), :], vv_ref, copy_sems.at[1])
        kcopy.start()
        vcopy.start()

        barrier = pltpu.get_barrier_semaphore()
        for j in range(N_DEV):
            @pl.when(j != me)
            def _():
                pl.semaphore_signal(
                    barrier, inc=1, device_id=(j,),
                    device_id_type=pl.DeviceIdType.MESH)
        pl.semaphore_wait(barrier, N_DEV - 1)

        xv = x_ref[...].reshape(ROWS, D).astype(jnp.bfloat16)
        q = jnp.dot(xv, wq_ref[...].astype(jnp.bfloat16),
                    preferred_element_type=jnp.float32)

        kcopy.wait()
        vcopy.wait()

        for b in range(B):
            for h in range(HQ):
                qbh = q[b*SQ:(b+1)*SQ, h*DH:(h+1)*DH].astype(jnp.bfloat16)
                kbh = kv_ref[b, :, h, :].astype(jnp.bfloat16)
                vbh = vv_ref[b, :, h, :].astype(jnp.bfloat16)
                s = lax.dot_general(
                    qbh, kbh, (((1,), (1,)), ((), ())),
                    preferred_element_type=jnp.float32) * 0.125
                m = jnp.max(s, axis=1, keepdims=True)
                p = jnp.exp(s - m)
                l = jnp.sum(p, axis=1, keepdims=True)
                o = lax.dot_general(
                    p.astype(jnp.bfloat16), vbh, (((1,), (0,)), ((), ())),
                    preferred_element_type=jnp.float32)
                attn_ref[b*SQ:(b+1)*SQ, h*DH:(h+1)*DH] = (
                    (o / l).astype(jnp.bfloat16))

        part_ref[...] = jnp.dot(
            attn_ref[...], wo_ref[...].astype(jnp.bfloat16),
            preferred_element_type=jnp.float32)

        p1 = []
        for j in range(N_DEV):
            rdma = pltpu.make_async_remote_copy(
                src_ref=part_ref.at[pl.ds(SEG * j, SEG)],
                dst_ref=rs_ref.at[me],
                send_sem=s1_send.at[j],
                recv_sem=s1_recv.at[me],
                device_id=(j,),
                device_id_type=pl.DeviceIdType.MESH)
            @pl.when(j != me)
            def _(rdma=rdma):
                rdma.start()
            p1.append(rdma)
        self1 = pltpu.make_async_copy(
            part_ref.at[pl.ds(pl.multiple_of(SEG * me, SEG), SEG)],
            rs_ref.at[me], copy_sems.at[2])
        self1.start()

        for s in range(N_DEV):
            recv = pltpu.make_async_remote_copy(
                src_ref=part_ref.at[pl.ds(0, SEG)],
                dst_ref=rs_ref.at[s],
                send_sem=s1_send.at[s],
                recv_sem=s1_recv.at[s],
                device_id=(0,),
                device_id_type=pl.DeviceIdType.MESH)
            @pl.when(s != me)
            def _(recv=recv):
                recv.wait_recv()
        self1.wait()

        stage_ref[...] = jnp.sum(rs_ref[...], axis=0)

        my_b = me // (SQ // SEG)
        my_r = pl.multiple_of((me % (SQ // SEG)) * SEG, SEG)
        p2 = []
        for j in range(N_DEV):
            rdma = pltpu.make_async_remote_copy(
                src_ref=stage_ref,
                dst_ref=out_ref.at[my_b, pl.ds(my_r, SEG)],
                send_sem=s2_send.at[j],
                recv_sem=s2_recv.at[me],
                device_id=(j,),
                device_id_type=pl.DeviceIdType.MESH)
            @pl.when(j != me)
            def _(rdma=rdma):
                rdma.start()
            p2.append(rdma)
        self2 = pltpu.make_async_copy(
            stage_ref, out_ref.at[my_b, pl.ds(my_r, SEG)], copy_sems.at[3])
        self2.start()

        for j, rdma in enumerate(p1):
            @pl.when(j != me)
            def _(rdma=rdma):
                rdma.wait_send()

        for s in range(N_DEV):
            recv = pltpu.make_async_remote_copy(
                src_ref=stage_ref,
                dst_ref=out_ref.at[s // (SQ // SEG),
                                   pl.ds((s % (SQ // SEG)) * SEG, SEG)],
                send_sem=s2_send.at[s],
                recv_sem=s2_recv.at[s],
                device_id=(0,),
                device_id_type=pl.DeviceIdType.MESH)
            @pl.when(s != me)
            def _(recv=recv):
                recv.wait_recv()
        self2.wait()

        for j, rdma in enumerate(p2):
            @pl.when(j != me)
            def _(rdma=rdma):
                rdma.wait_send()

    return pl.pallas_call(
        body,
        out_shape=jax.ShapeDtypeStruct((B, SQ, D), jnp.float32),
        in_specs=[
            pl.BlockSpec(memory_space=pltpu.VMEM),
            pl.BlockSpec(memory_space=pltpu.VMEM),
            pl.BlockSpec(memory_space=pltpu.VMEM),
            pl.BlockSpec(memory_space=pl.ANY),
            pl.BlockSpec(memory_space=pl.ANY),
        ],
        out_specs=pl.BlockSpec(memory_space=pltpu.VMEM),
        scratch_shapes=[
            pltpu.VMEM((ROWS, D), jnp.float32),
            pltpu.VMEM((N_DEV, SEG, D), jnp.float32),
            pltpu.VMEM((SEG, D), jnp.float32),
            pltpu.VMEM((B, SKV, HQ, DH), jnp.float32),
            pltpu.VMEM((B, SKV, HQ, DH), jnp.float32),
            pltpu.VMEM((ROWS, D), jnp.bfloat16),
            pltpu.SemaphoreType.DMA((N_DEV,)),
            pltpu.SemaphoreType.DMA((N_DEV,)),
            pltpu.SemaphoreType.DMA((N_DEV,)),
            pltpu.SemaphoreType.DMA((N_DEV,)),
            pltpu.SemaphoreType.DMA((4,)),
        ],
        compiler_params=pltpu.CompilerParams(collective_id=0),
    )(x, Wq, Wo, K_ext, V_ext)


# device time: 42576 ns/iter; 2.0945x vs baseline; 1.3845x over previous
import jax
import jax.numpy as jnp
from jax import lax
from jax.experimental import pallas as pl
from jax.experimental.pallas import tpu as pltpu

N_DEV = 32
B, SQ, D = 2, 128, 512
HQ, DH = 8, 64
SKV = 128
ROWS = B * SQ
SEG = ROWS // N_DEV


def kernel(x, Wq, Wo, K_ext, V_ext):
    pos = lax.axis_index("i")
    K_sl = lax.dynamic_slice_in_dim(K_ext, pos * HQ, HQ, axis=2)
    V_sl = lax.dynamic_slice_in_dim(V_ext, pos * HQ, HQ, axis=2)

    def body(x_ref, wq_ref, wo_ref, kv_ref, vv_ref, out_ref,
             part_ref, rs_ref, stage_ref, attn_ref,
             s1_send, s1_recv, s2_send, s2_recv, copy_sems):
        me = lax.axis_index("i")

        barrier = pltpu.get_barrier_semaphore()
        for j in range(N_DEV):
            @pl.when(j != me)
            def _():
                pl.semaphore_signal(
                    barrier, inc=1, device_id=(j,),
                    device_id_type=pl.DeviceIdType.MESH)

        xv = x_ref[...].reshape(ROWS, D).astype(jnp.bfloat16)
        q = jnp.dot(xv, wq_ref[...].astype(jnp.bfloat16),
                    preferred_element_type=jnp.float32)

        for b in range(B):
            for h in range(HQ):
                qbh = q[b*SQ:(b+1)*SQ, h*DH:(h+1)*DH].astype(jnp.bfloat16)
                kbh = kv_ref[b, :, h, :].astype(jnp.bfloat16)
                vbh = vv_ref[b, :, h, :].astype(jnp.bfloat16)
                s = lax.dot_general(
                    qbh, kbh, (((1,), (1,)), ((), ())),
                    preferred_element_type=jnp.float32) * 0.125
                m = jnp.max(s, axis=1, keepdims=True)
                p = jnp.exp(s - m)
                l = jnp.sum(p, axis=1, keepdims=True)
                o = lax.dot_general(
                    p.astype(jnp.bfloat16), vbh, (((1,), (0,)), ((), ())),
                    preferred_element_type=jnp.float32)
                attn_ref[b*SQ:(b+1)*SQ, h*DH:(h+1)*DH] = (
                    (o / l).astype(jnp.bfloat16))

        part_ref[...] = jnp.dot(
            attn_ref[...], wo_ref[...].astype(jnp.bfloat16),
            preferred_element_type=jnp.float32)

        pl.semaphore_wait(barrier, N_DEV - 1)

        p1 = []
        for j in range(N_DEV):
            rdma = pltpu.make_async_remote_copy(
                src_ref=part_ref.at[pl.ds(SEG * j, SEG)],
                dst_ref=rs_ref.at[me],
                send_sem=s1_send.at[j],
                recv_sem=s1_recv.at[me],
                device_id=(j,),
                device_id_type=pl.DeviceIdType.MESH)
            @pl.when(j != me)
            def _(rdma=rdma):
                rdma.start()
            p1.append(rdma)
        self1 = pltpu.make_async_copy(
            part_ref.at[pl.ds(pl.multiple_of(SEG * me, SEG), SEG)],
            rs_ref.at[me], copy_sems.at[0])
        self1.start()

        for s in range(N_DEV):
            recv = pltpu.make_async_remote_copy(
                src_ref=part_ref.at[pl.ds(0, SEG)],
                dst_ref=rs_ref.at[s],
                send_sem=s1_send.at[s],
                recv_sem=s1_recv.at[s],
                device_id=(0,),
                device_id_type=pl.DeviceIdType.MESH)
            @pl.when(s != me)
            def _(recv=recv):
                recv.wait_recv()
        self1.wait()

        stage_ref[...] = jnp.sum(rs_ref[...], axis=0)

        my_b = me // (SQ // SEG)
        my_r = pl.multiple_of((me % (SQ // SEG)) * SEG, SEG)
        p2 = []
        for j in range(N_DEV):
            rdma = pltpu.make_async_remote_copy(
                src_ref=stage_ref,
                dst_ref=out_ref.at[my_b, pl.ds(my_r, SEG)],
                send_sem=s2_send.at[j],
                recv_sem=s2_recv.at[me],
                device_id=(j,),
                device_id_type=pl.DeviceIdType.MESH)
            @pl.when(j != me)
            def _(rdma=rdma):
                rdma.start()
            p2.append(rdma)
        self2 = pltpu.make_async_copy(
            stage_ref, out_ref.at[my_b, pl.ds(my_r, SEG)], copy_sems.at[1])
        self2.start()

        for j, rdma in enumerate(p1):
            @pl.when(j != me)
            def _(rdma=rdma):
                rdma.wait_send()

        for s in range(N_DEV):
            recv = pltpu.make_async_remote_copy(
                src_ref=stage_ref,
                dst_ref=out_ref.at[s // (SQ // SEG),
                                   pl.ds((s % (SQ // SEG)) * SEG, SEG)],
                send_sem=s2_send.at[s],
                recv_sem=s2_recv.at[s],
                device_id=(0,),
                device_id_type=pl.DeviceIdType.MESH)
            @pl.when(s != me)
            def _(recv=recv):
                recv.wait_recv()
        self2.wait()

        for j, rdma in enumerate(p2):
            @pl.when(j != me)
            def _(rdma=rdma):
                rdma.wait_send()

    return pl.pallas_call(
        body,
        out_shape=jax.ShapeDtypeStruct((B, SQ, D), jnp.float32),
        in_specs=[
            pl.BlockSpec(memory_space=pltpu.VMEM),
            pl.BlockSpec(memory_space=pltpu.VMEM),
            pl.BlockSpec(memory_space=pltpu.VMEM),
            pl.BlockSpec(memory_space=pltpu.VMEM),
            pl.BlockSpec(memory_space=pltpu.VMEM),
        ],
        out_specs=pl.BlockSpec(memory_space=pltpu.VMEM),
        scratch_shapes=[
            pltpu.VMEM((ROWS, D), jnp.float32),
            pltpu.VMEM((N_DEV, SEG, D), jnp.float32),
            pltpu.VMEM((SEG, D), jnp.float32),
            pltpu.VMEM((ROWS, D), jnp.bfloat16),
            pltpu.SemaphoreType.DMA((N_DEV,)),
            pltpu.SemaphoreType.DMA((N_DEV,)),
            pltpu.SemaphoreType.DMA((N_DEV,)),
            pltpu.SemaphoreType.DMA((N_DEV,)),
            pltpu.SemaphoreType.DMA((2,)),
        ],
        compiler_params=pltpu.CompilerParams(collective_id=0),
    )(x, Wq, Wo, K_sl, V_sl)
